# initial kernel scaffold (unmeasured)
import jax
import jax.numpy as jnp
from jax import lax
from jax.experimental import pallas as pl
from jax.experimental.pallas import tpu as pltpu


def kernel(
    x,
):
    def body(*refs):
        pass

    out_shape = jax.ShapeDtypeStruct(..., jnp.float32)
    return pl.pallas_call(body, out_shape=out_shape)(...)


# baseline (device time: 25630 ns/iter reference)
import jax
import jax.numpy as jnp
from jax import lax
from jax.experimental import pallas as pl
from jax.experimental.pallas import tpu as pltpu

N_DEV = 8


def kernel(x):
    m, n_per = x.shape

    def body(x_ref, out_ref, my_stats, peer_stats, send_sems, recv_sems):
        my = lax.axis_index("i")

        xf = x_ref[:, :].astype(jnp.float32)
        lmax = jnp.max(xf, axis=1, keepdims=True)
        e = jnp.exp(xf - lmax)
        lsum = jnp.sum(e, axis=1, keepdims=True)

        my_stats[:, 0:1] = lmax
        my_stats[:, 1:2] = lsum

        rdmas = []
        for d in range(1, N_DEV):
            tgt = (my + d) % N_DEV
            rdma = pltpu.make_async_remote_copy(
                src_ref=my_stats,
                dst_ref=peer_stats.at[d - 1],
                send_sem=send_sems.at[d - 1],
                recv_sem=recv_sems.at[d - 1],
                device_id=(tgt,),
                device_id_type=pl.DeviceIdType.MESH,
            )
            rdma.start()
            rdmas.append(rdma)
        for rdma in rdmas:
            rdma.wait()

        pmax = peer_stats[:, :, 0:1]
        psum = peer_stats[:, :, 1:2]
        gmax = jnp.maximum(lmax, jnp.max(pmax, axis=0))
        gsum = lsum * jnp.exp(lmax - gmax) + jnp.sum(
            psum * jnp.exp(pmax - gmax), axis=0
        )
        out_ref[:, :] = (e * (jnp.exp(lmax - gmax) / gsum)).astype(out_ref.dtype)

    return pl.pallas_call(
        body,
        out_shape=jax.ShapeDtypeStruct((m, n_per), x.dtype),
        in_specs=[pl.BlockSpec(memory_space=pltpu.VMEM)],
        out_specs=pl.BlockSpec(memory_space=pltpu.VMEM),
        scratch_shapes=[
            pltpu.VMEM((m, 2), jnp.float32),
            pltpu.VMEM((N_DEV - 1, m, 2), jnp.float32),
            pltpu.SemaphoreType.DMA((N_DEV - 1,)),
            pltpu.SemaphoreType.DMA((N_DEV - 1,)),
        ],
    )(x)


# device time: 8650 ns/iter; 2.9630x vs baseline; 2.9630x over previous
import jax
import jax.numpy as jnp
from jax import lax
from jax.experimental import pallas as pl
from jax.experimental.pallas import tpu as pltpu

N_DEV = 8


def kernel(x):
    m, n_per = x.shape

    def body(x_ref, out_ref, my_stats, peer_stats, send_sems, recv_sems):
        my = lax.axis_index("i")

        barrier_sem = pltpu.get_barrier_semaphore()
        for d in range(1, N_DEV):
            pl.semaphore_signal(
                barrier_sem,
                inc=1,
                device_id=((my + d) % N_DEV,),
                device_id_type=pl.DeviceIdType.MESH,
            )

        xf = x_ref[:, :].astype(jnp.float32)
        lmax = jnp.max(xf, axis=1, keepdims=True)
        e = jnp.exp(xf - lmax)
        lsum = jnp.sum(e, axis=1, keepdims=True)

        my_stats[0:1, :] = jnp.transpose(lmax)
        my_stats[1:2, :] = jnp.transpose(lsum)

        pl.semaphore_wait(barrier_sem, N_DEV - 1)

        rdmas = []
        for d in range(1, N_DEV):
            tgt = (my + d) % N_DEV
            rdma = pltpu.make_async_remote_copy(
                src_ref=my_stats,
                dst_ref=peer_stats.at[d - 1],
                send_sem=send_sems.at[d - 1],
                recv_sem=recv_sems.at[d - 1],
                device_id=(tgt,),
                device_id_type=pl.DeviceIdType.MESH,
            )
            rdma.start()
            rdmas.append(rdma)
        for rdma in rdmas:
            rdma.wait()

        pmax = peer_stats[:, 0:1, :]
        psum = peer_stats[:, 1:2, :]
        lmax_row = my_stats[0:1, :]
        lsum_row = my_stats[1:2, :]
        gmax = jnp.maximum(lmax_row, jnp.max(pmax, axis=0))
        gsum = lsum_row * jnp.exp(lmax_row - gmax) + jnp.sum(
            psum * jnp.exp(pmax - gmax), axis=0
        )
        scale = jnp.exp(lmax_row - gmax) / gsum
        out_ref[:, :] = (e * jnp.transpose(scale)).astype(out_ref.dtype)

    return pl.pallas_call(
        body,
        out_shape=jax.ShapeDtypeStruct((m, n_per), x.dtype),
        in_specs=[pl.BlockSpec(memory_space=pltpu.VMEM)],
        out_specs=pl.BlockSpec(memory_space=pltpu.VMEM),
        scratch_shapes=[
            pltpu.VMEM((2, m), jnp.float32),
            pltpu.VMEM((N_DEV - 1, 2, m), jnp.float32),
            pltpu.SemaphoreType.DMA((N_DEV - 1,)),
            pltpu.SemaphoreType.DMA((N_DEV - 1,)),
        ],
        compiler_params=pltpu.CompilerParams(collective_id=0),
    )(x)


# device time: 8642 ns/iter; 2.9657x vs baseline; 1.0009x over previous
import jax
import jax.numpy as jnp
from jax import lax
from jax.experimental import pallas as pl
from jax.experimental.pallas import tpu as pltpu

N_DEV = 8


def kernel(x):
    m, n_per = x.shape

    def body(x_ref, out_ref, my_stats, peer_stats, send_sems, recv_sems):
        my = lax.axis_index("i")

        barrier_sem = pltpu.get_barrier_semaphore()
        for d in range(1, N_DEV):
            pl.semaphore_signal(
                barrier_sem,
                inc=1,
                device_id=((my + d) % N_DEV,),
                device_id_type=pl.DeviceIdType.MESH,
            )

        xf = x_ref[:, :].astype(jnp.float32)
        lmax = jnp.max(xf, axis=1, keepdims=True)
        e = jnp.exp(xf - lmax)
        lsum = jnp.sum(e, axis=1, keepdims=True)

        my_stats[0:1, :] = jnp.transpose(lmax)
        my_stats[1:2, :] = jnp.transpose(lsum)

        pl.semaphore_wait(barrier_sem, N_DEV - 1)

        rdmas = []
        for d in range(1, N_DEV):
            rdma = pltpu.make_async_remote_copy(
                src_ref=my_stats,
                dst_ref=peer_stats.at[d - 1],
                send_sem=send_sems.at[d - 1],
                recv_sem=recv_sems.at[d - 1],
                device_id=((my + d) % N_DEV,),
                device_id_type=pl.DeviceIdType.MESH,
            )
            rdma.start()
            rdmas.append(rdma)

        for rdma in rdmas:
            rdma.wait_recv()

        pmax = peer_stats[:, 0:1, :]
        psum = peer_stats[:, 1:2, :]
        lmax_row = my_stats[0:1, :]
        lsum_row = my_stats[1:2, :]
        gmax = jnp.maximum(lmax_row, jnp.max(pmax, axis=0))
        gsum = lsum_row * jnp.exp(lmax_row - gmax) + jnp.sum(
            psum * jnp.exp(pmax - gmax), axis=0
        )
        scale = jnp.exp(lmax_row - gmax) / gsum
        out_ref[:, :] = (e * jnp.transpose(scale)).astype(out_ref.dtype)

        for rdma in rdmas:
            rdma.wait_send()

    return pl.pallas_call(
        body,
        out_shape=jax.ShapeDtypeStruct((m, n_per), x.dtype),
        in_specs=[pl.BlockSpec(memory_space=pltpu.VMEM)],
        out_specs=pl.BlockSpec(memory_space=pltpu.VMEM),
        scratch_shapes=[
            pltpu.VMEM((2, m), jnp.float32),
            pltpu.VMEM((N_DEV - 1, 2, m), jnp.float32),
            pltpu.SemaphoreType.DMA((N_DEV - 1,)),
            pltpu.SemaphoreType.DMA((N_DEV - 1,)),
        ],
        compiler_params=pltpu.CompilerParams(collective_id=0),
    )(x)
